# baseline (speedup 1.0000x reference)
import functools
import math

import jax
import jax.numpy as jnp
from jax.experimental import pallas as pl
from jax.experimental.pallas import tpu as pltpu

D = 1024
H = 16
HK = 4
DH = 64
E = 16
F = 768
B = 32
S = 4
T = B * S
EPS = 1e-06
THETA = 1000000.0
TILE = 16
TP = 256
NBUF = 4


def _rms(x, w, eps=EPS):
    var = jnp.mean(x * x, axis=-1, keepdims=True)
    return x * jax.lax.rsqrt(var + eps) * w


def _fused_body(h_ref, ln1_ref, wq_hbm, wk_hbm, wv_hbm, qn_ref, kn_ref,
                wo_hbm, ln2_ref, rw_hbm, wgu_hbm, wd_hbm, out_ref,
                x2_scr, p_scr, off_scr, cnt_scr, y_scr,
                wq_s, wk_s, wv_s, wo_s, rw_s,
                gu_buf, d_buf, a_sem, gu_sem, d_sem):
    i = pl.program_id(0)

    def _gu_copy(idx, s, q):
        rows = pl.ds(q * (D // 4), D // 4)
        return pltpu.make_async_copy(wgu_hbm.at[idx, rows, :],
                                     gu_buf.at[s, rows, :],
                                     gu_sem.at[s, q])

    def _d_copy(idx, s, half):
        rows = pl.ds(half * (F // 2), F // 2)
        return pltpu.make_async_copy(wd_hbm.at[idx, rows, :],
                                     d_buf.at[s, rows, :],
                                     d_sem.at[s, half])

    def start_copy(idx, s):
        for q in range(4):
            _gu_copy(idx, s, q).start()
        for half in range(2):
            _d_copy(idx, s, half).start()

    _attn_copies = lambda: [
        pltpu.make_async_copy(wq_hbm, wq_s, a_sem.at[0]),
        pltpu.make_async_copy(wk_hbm, wk_s, a_sem.at[1]),
        pltpu.make_async_copy(wv_hbm, wv_s, a_sem.at[2]),
        pltpu.make_async_copy(wo_hbm, wo_s, a_sem.at[3]),
        pltpu.make_async_copy(rw_hbm, rw_s, a_sem.at[4]),
    ]

    @pl.when(i == 0)
    def _attn_router():
        for c in _attn_copies():
            c.start()
        for b in range(NBUF):
            start_copy(b, b)
        y_scr[...] = jnp.zeros_like(y_scr)
        for c in _attn_copies():
            c.wait()

        h = h_ref[...].reshape(T, D)
        x = _rms(h, ln1_ref[...])
        q2 = jnp.dot(x, wq_s[...], preferred_element_type=jnp.float32)
        k2 = jnp.dot(x, wk_s[...], preferred_element_type=jnp.float32)
        v2 = jnp.dot(x, wv_s[...], preferred_element_type=jnp.float32)
        rowpos = jax.lax.broadcasted_iota(jnp.int32, (T, DH), 0) % S
        colf = jax.lax.broadcasted_iota(jnp.int32, (T, DH), 1) % (DH // 2)
        inv = jnp.exp(colf.astype(jnp.float32) *
                      jnp.float32(-2.0 * math.log(THETA) / DH))
        ang = rowpos.astype(jnp.float32) * inv
        cos = jnp.cos(ang)
        sin = jnp.sin(ang)
        qn = qn_ref[...]
        kn = kn_ref[...]

        def rope(z):
            zr = jnp.concatenate([-z[:, DH // 2:], z[:, :DH // 2]], axis=1)
            return z * cos + zr * sin

        ks = []
        vs = []
        for j in range(HK):
            kj = rope(_rms(k2[:, j * DH:(j + 1) * DH], kn))
            ks.append(kj)
            vs.append(v2[:, j * DH:(j + 1) * DH])

        ti = jax.lax.broadcasted_iota(jnp.int32, (T, T), 0)
        tj = jax.lax.broadcasted_iota(jnp.int32, (T, T), 1)
        mask = (ti // S == tj // S) & (tj % S <= ti % S)
        neg = jnp.float32(-1e30)
        scale = jnp.float32(1.0 / math.sqrt(DH))

        outs = []
        for hh in range(H):
            j = hh // (H // HK)
            qh = rope(_rms(q2[:, hh * DH:(hh + 1) * DH], qn))
            sc = jax.lax.dot_general(qh, ks[j], (((1,), (1,)), ((), ())),
                                     preferred_element_type=jnp.float32)
            sc = jnp.where(mask, sc * scale, neg)
            ex = jnp.exp(sc - jnp.max(sc, axis=1, keepdims=True))
            at = ex / jnp.sum(ex, axis=1, keepdims=True)
            outs.append(jnp.dot(at, vs[j], preferred_element_type=jnp.float32))
        o2 = jnp.concatenate(outs, axis=1)
        hidden = h + jnp.dot(o2, wo_s[...],
                             preferred_element_type=jnp.float32)
        out_ref[...] = hidden.reshape(B, S, D)

        x2 = _rms(hidden, ln2_ref[...])
        logits = jnp.dot(x2, rw_s[...], preferred_element_type=jnp.float32)

        eidx = jax.lax.broadcasted_iota(jnp.int32, (T, E), 1)
        rowmax = jnp.max(logits, axis=1, keepdims=True)
        assign = jnp.min(jnp.where(logits == rowmax, eidx, E), axis=1,
                         keepdims=True)
        onehot = (eidx == assign).astype(jnp.float32)
        counts = jnp.sum(onehot, axis=0, keepdims=True)
        padded = jnp.float32(8.0) * jnp.ceil(counts * jnp.float32(1.0 / 8.0))
        er = jax.lax.broadcasted_iota(jnp.int32, (E, E), 0)
        ec = jax.lax.broadcasted_iota(jnp.int32, (E, E), 1)
        lt_e = (er < ec).astype(jnp.float32)
        offsets = jnp.dot(padded, lt_e,
                          preferred_element_type=jnp.float32)
        lt_t = (tj < ti).astype(jnp.float32)
        intra = jnp.dot(lt_t, onehot,
                        preferred_element_type=jnp.float32)
        rank = jnp.sum(onehot * (offsets + intra), axis=1,
                       keepdims=True)
        rj = jax.lax.broadcasted_iota(jnp.int32, (T, TP), 1)
        p_mat = (rank.astype(jnp.int32) == rj).astype(jnp.float32)
        p_scr[...] = p_mat
        x2_scr[...] = jnp.dot(p_mat.T, x2, preferred_element_type=jnp.float32)
        se = jax.lax.broadcasted_iota(jnp.int32, (E, E * 128), 0)
        sl = jax.lax.broadcasted_iota(jnp.int32, (E, E * 128), 1)
        spread = (sl // 128 == se).astype(jnp.float32)
        off_scr[...] = jnp.dot(offsets * jnp.float32(1.0 / 8.0), spread,
                               preferred_element_type=jnp.float32
                               ).astype(jnp.int32)
        cnt_scr[...] = jnp.dot(counts, spread,
                               preferred_element_type=jnp.float32
                               ).astype(jnp.int32)

    @pl.when(i > 0)
    def _expert():
        e = i - 1
        slot = jax.lax.rem(e, NBUF)
        off8 = off_scr[0, e * 128]
        cnt = cnt_scr[0, e * 128]
        for q in range(4):
            _gu_copy(e, slot, q).wait()
        for half in range(2):
            _d_copy(e, slot, half).wait()

        ntiles = (cnt + TILE - 1) // TILE
        wgu = gu_buf[slot]
        wd = d_buf[slot]
        rid = jax.lax.broadcasted_iota(jnp.int32, (TILE, 1), 0)

        def tile_step(t, _):
            base = off8 * 8 + t * TILE
            xb = x2_scr[pl.ds(base, TILE), :]
            gu = jnp.dot(xb, wgu, preferred_element_type=jnp.float32)
            gate = gu[:, :F]
            up = gu[:, F:]
            act = gate * jax.lax.logistic(gate) * up
            y = jnp.dot(act, wd, preferred_element_type=jnp.float32)
            valid = t * TILE + rid < cnt
            cur = y_scr[pl.ds(base, TILE), :]
            y_scr[pl.ds(base, TILE), :] = jnp.where(valid, y, cur)
            return 0

        jax.lax.fori_loop(0, ntiles, tile_step, 0)

        @pl.when(e + NBUF < E)
        def _next():
            start_copy(e + NBUF, slot)

    @pl.when(i == E)
    def _finish():
        moe = jnp.dot(p_scr[...], y_scr[...],
                      preferred_element_type=jnp.float32)
        out_ref[...] = out_ref[...] + moe.reshape(B, S, D)


@jax.jit
def kernel(hidden_states, ln1_w, wq, wk, wv, q_norm_w, k_norm_w, wo, ln2_w,
           router_w, w_gate_up, w_down):
    whole = lambda e: (0,) * 2
    out = pl.pallas_call(
        _fused_body,
        grid=(E + 1,),
        in_specs=[
            pl.BlockSpec((B, S, D), lambda e: (0, 0, 0)),
            pl.BlockSpec((1, D), whole),
            pl.BlockSpec(memory_space=pltpu.MemorySpace.HBM),
            pl.BlockSpec(memory_space=pltpu.MemorySpace.HBM),
            pl.BlockSpec(memory_space=pltpu.MemorySpace.HBM),
            pl.BlockSpec((1, DH), whole),
            pl.BlockSpec((1, DH), whole),
            pl.BlockSpec(memory_space=pltpu.MemorySpace.HBM),
            pl.BlockSpec((1, D), whole),
            pl.BlockSpec(memory_space=pltpu.MemorySpace.HBM),
            pl.BlockSpec(memory_space=pltpu.MemorySpace.HBM),
            pl.BlockSpec(memory_space=pltpu.MemorySpace.HBM),
        ],
        out_specs=pl.BlockSpec((B, S, D), lambda e: (0, 0, 0)),
        scratch_shapes=[
            pltpu.VMEM((TP, D), jnp.float32),
            pltpu.VMEM((T, TP), jnp.float32),
            pltpu.VMEM((1, E * 128), jnp.int32),
            pltpu.VMEM((1, E * 128), jnp.int32),
            pltpu.VMEM((TP, D), jnp.float32),
            pltpu.VMEM((D, H * DH), jnp.float32),
            pltpu.VMEM((D, HK * DH), jnp.float32),
            pltpu.VMEM((D, HK * DH), jnp.float32),
            pltpu.VMEM((H * DH, D), jnp.float32),
            pltpu.VMEM((D, E), jnp.float32),
            pltpu.VMEM((NBUF, D, 2 * F), jnp.float32),
            pltpu.VMEM((NBUF, F, D), jnp.float32),
            pltpu.SemaphoreType.DMA((5,)),
            pltpu.SemaphoreType.DMA((NBUF, 4)),
            pltpu.SemaphoreType.DMA((NBUF, 2)),
        ],
        out_shape=jax.ShapeDtypeStruct((B, S, D), jnp.float32),
    )(hidden_states, ln1_w.reshape(1, D), wq, wk, wv,
      q_norm_w.reshape(1, DH), k_norm_w.reshape(1, DH), wo,
      ln2_w.reshape(1, D), router_w, w_gate_up, w_down)
    return out


# speedup vs baseline: 1.0100x; 1.0100x over previous
import functools
import math

import jax
import jax.numpy as jnp
from jax.experimental import pallas as pl
from jax.experimental.pallas import tpu as pltpu

D = 1024
H = 16
HK = 4
DH = 64
E = 16
F = 768
B = 32
S = 4
T = B * S
EPS = 1e-06
THETA = 1000000.0
TILE = 16
TP = 256
NBUF = 4


def _rms(x, w, eps=EPS):
    var = jnp.mean(x * x, axis=-1, keepdims=True)
    return x * jax.lax.rsqrt(var + eps) * w


def _fused_body(h_ref, ln1_ref, wq_hbm, wk_hbm, wv_hbm, qn_ref, kn_ref,
                wo_hbm, ln2_ref, rw_hbm, wgu_hbm, wd_hbm, out_ref,
                x2_scr, p_scr, off_scr, cnt_scr, y_scr,
                wq_s, wk_s, wv_s, wo_s, rw_s,
                gu_buf, d_buf, a_sem, gu_sem, d_sem):
    i = pl.program_id(0)

    def _gu_copy(idx, s, q):
        rows = pl.ds(q * (D // 2), D // 2)
        return pltpu.make_async_copy(wgu_hbm.at[idx, rows, :],
                                     gu_buf.at[s, rows, :],
                                     gu_sem.at[s, q])

    def _d_copy(idx, s, half):
        rows = pl.ds(half * (F // 2), F // 2)
        return pltpu.make_async_copy(wd_hbm.at[idx, rows, :],
                                     d_buf.at[s, rows, :],
                                     d_sem.at[s, half])

    def start_copy(idx, s):
        for q in range(2):
            _gu_copy(idx, s, q).start()
        for half in range(2):
            _d_copy(idx, s, half).start()

    _attn_copies = lambda: [
        pltpu.make_async_copy(wq_hbm, wq_s, a_sem.at[0]),
        pltpu.make_async_copy(wk_hbm, wk_s, a_sem.at[1]),
        pltpu.make_async_copy(wv_hbm, wv_s, a_sem.at[2]),
        pltpu.make_async_copy(wo_hbm, wo_s, a_sem.at[3]),
        pltpu.make_async_copy(rw_hbm, rw_s, a_sem.at[4]),
    ]

    @pl.when(i == 0)
    def _attn_router():
        for c in _attn_copies():
            c.start()
        for b in range(NBUF):
            start_copy(b, b)
        y_scr[...] = jnp.zeros_like(y_scr)
        for c in _attn_copies():
            c.wait()

        h = h_ref[...].reshape(T, D)
        x = _rms(h, ln1_ref[...])
        q2 = jnp.dot(x, wq_s[...], preferred_element_type=jnp.float32)
        k2 = jnp.dot(x, wk_s[...], preferred_element_type=jnp.float32)
        v2 = jnp.dot(x, wv_s[...], preferred_element_type=jnp.float32)
        rowpos = jax.lax.broadcasted_iota(jnp.int32, (T, DH), 0) % S
        colf = jax.lax.broadcasted_iota(jnp.int32, (T, DH), 1) % (DH // 2)
        inv = jnp.exp(colf.astype(jnp.float32) *
                      jnp.float32(-2.0 * math.log(THETA) / DH))
        ang = rowpos.astype(jnp.float32) * inv
        cos = jnp.cos(ang)
        sin = jnp.sin(ang)
        qn = qn_ref[...]
        kn = kn_ref[...]

        def rope(z):
            zr = jnp.concatenate([-z[:, DH // 2:], z[:, :DH // 2]], axis=1)
            return z * cos + zr * sin

        ks = []
        vs = []
        for j in range(HK):
            kj = rope(_rms(k2[:, j * DH:(j + 1) * DH], kn))
            ks.append(kj)
            vs.append(v2[:, j * DH:(j + 1) * DH])

        ti = jax.lax.broadcasted_iota(jnp.int32, (T, T), 0)
        tj = jax.lax.broadcasted_iota(jnp.int32, (T, T), 1)
        mask = (ti // S == tj // S) & (tj % S <= ti % S)
        neg = jnp.float32(-1e30)
        scale = jnp.float32(1.0 / math.sqrt(DH))

        outs = []
        for hh in range(H):
            j = hh // (H // HK)
            qh = rope(_rms(q2[:, hh * DH:(hh + 1) * DH], qn))
            sc = jax.lax.dot_general(qh, ks[j], (((1,), (1,)), ((), ())),
                                     preferred_element_type=jnp.float32)
            sc = jnp.where(mask, sc * scale, neg)
            ex = jnp.exp(sc - jnp.max(sc, axis=1, keepdims=True))
            at = ex / jnp.sum(ex, axis=1, keepdims=True)
            outs.append(jnp.dot(at, vs[j], preferred_element_type=jnp.float32))
        o2 = jnp.concatenate(outs, axis=1)
        hidden = h + jnp.dot(o2, wo_s[...],
                             preferred_element_type=jnp.float32)
        out_ref[...] = hidden.reshape(B, S, D)

        x2 = _rms(hidden, ln2_ref[...])
        logits = jnp.dot(x2, rw_s[...], preferred_element_type=jnp.float32)

        eidx = jax.lax.broadcasted_iota(jnp.int32, (T, E), 1)
        rowmax = jnp.max(logits, axis=1, keepdims=True)
        assign = jnp.min(jnp.where(logits == rowmax, eidx, E), axis=1,
                         keepdims=True)
        onehot = (eidx == assign).astype(jnp.float32)
        counts = jnp.sum(onehot, axis=0, keepdims=True)
        padded = jnp.float32(8.0) * jnp.ceil(counts * jnp.float32(1.0 / 8.0))
        er = jax.lax.broadcasted_iota(jnp.int32, (E, E), 0)
        ec = jax.lax.broadcasted_iota(jnp.int32, (E, E), 1)
        lt_e = (er < ec).astype(jnp.float32)
        offsets = jnp.dot(padded, lt_e,
                          preferred_element_type=jnp.float32)
        lt_t = (tj < ti).astype(jnp.float32)
        intra = jnp.dot(lt_t, onehot,
                        preferred_element_type=jnp.float32)
        rank = jnp.sum(onehot * (offsets + intra), axis=1,
                       keepdims=True)
        rj = jax.lax.broadcasted_iota(jnp.int32, (T, TP), 1)
        p_mat = (rank.astype(jnp.int32) == rj).astype(jnp.float32)
        p_scr[...] = p_mat
        x2_scr[...] = jnp.dot(p_mat.T, x2, preferred_element_type=jnp.float32)
        se = jax.lax.broadcasted_iota(jnp.int32, (E, E * 128), 0)
        sl = jax.lax.broadcasted_iota(jnp.int32, (E, E * 128), 1)
        spread = (sl // 128 == se).astype(jnp.float32)
        off_scr[...] = jnp.dot(offsets * jnp.float32(1.0 / 8.0), spread,
                               preferred_element_type=jnp.float32
                               ).astype(jnp.int32)
        cnt_scr[...] = jnp.dot(counts, spread,
                               preferred_element_type=jnp.float32
                               ).astype(jnp.int32)

    @pl.when(i > 0)
    def _expert():
        e = i - 1
        slot = jax.lax.rem(e, NBUF)
        off8 = off_scr[0, e * 128]
        cnt = cnt_scr[0, e * 128]
        for q in range(2):
            _gu_copy(e, slot, q).wait()
        for half in range(2):
            _d_copy(e, slot, half).wait()

        ntiles = (cnt + TILE - 1) // TILE
        wgu = gu_buf[slot]
        wd = d_buf[slot]
        rid = jax.lax.broadcasted_iota(jnp.int32, (TILE, 1), 0)

        def tile_step(t, _):
            base = off8 * 8 + t * TILE
            xb = x2_scr[pl.ds(base, TILE), :]
            gu = jnp.dot(xb, wgu, preferred_element_type=jnp.float32)
            gate = gu[:, :F]
            up = gu[:, F:]
            act = gate * jax.lax.logistic(gate) * up
            y = jnp.dot(act, wd, preferred_element_type=jnp.float32)
            valid = t * TILE + rid < cnt
            cur = y_scr[pl.ds(base, TILE), :]
            y_scr[pl.ds(base, TILE), :] = jnp.where(valid, y, cur)
            return 0

        jax.lax.fori_loop(0, ntiles, tile_step, 0)

        @pl.when(e + NBUF < E)
        def _next():
            start_copy(e + NBUF, slot)

    @pl.when(i == E)
    def _finish():
        moe = jnp.dot(p_scr[...], y_scr[...],
                      preferred_element_type=jnp.float32)
        out_ref[...] = out_ref[...] + moe.reshape(B, S, D)


@jax.jit
def kernel(hidden_states, ln1_w, wq, wk, wv, q_norm_w, k_norm_w, wo, ln2_w,
           router_w, w_gate_up, w_down):
    whole = lambda e: (0,) * 2
    out = pl.pallas_call(
        _fused_body,
        grid=(E + 1,),
        in_specs=[
            pl.BlockSpec((B, S, D), lambda e: (0, 0, 0)),
            pl.BlockSpec((1, D), whole),
            pl.BlockSpec(memory_space=pltpu.MemorySpace.HBM),
            pl.BlockSpec(memory_space=pltpu.MemorySpace.HBM),
            pl.BlockSpec(memory_space=pltpu.MemorySpace.HBM),
            pl.BlockSpec((1, DH), whole),
            pl.BlockSpec((1, DH), whole),
            pl.BlockSpec(memory_space=pltpu.MemorySpace.HBM),
            pl.BlockSpec((1, D), whole),
            pl.BlockSpec(memory_space=pltpu.MemorySpace.HBM),
            pl.BlockSpec(memory_space=pltpu.MemorySpace.HBM),
            pl.BlockSpec(memory_space=pltpu.MemorySpace.HBM),
        ],
        out_specs=pl.BlockSpec((B, S, D), lambda e: (0, 0, 0)),
        scratch_shapes=[
            pltpu.VMEM((TP, D), jnp.float32),
            pltpu.VMEM((T, TP), jnp.float32),
            pltpu.VMEM((1, E * 128), jnp.int32),
            pltpu.VMEM((1, E * 128), jnp.int32),
            pltpu.VMEM((TP, D), jnp.float32),
            pltpu.VMEM((D, H * DH), jnp.float32),
            pltpu.VMEM((D, HK * DH), jnp.float32),
            pltpu.VMEM((D, HK * DH), jnp.float32),
            pltpu.VMEM((H * DH, D), jnp.float32),
            pltpu.VMEM((D, E), jnp.float32),
            pltpu.VMEM((NBUF, D, 2 * F), jnp.float32),
            pltpu.VMEM((NBUF, F, D), jnp.float32),
            pltpu.SemaphoreType.DMA((5,)),
            pltpu.SemaphoreType.DMA((NBUF, 2)),
            pltpu.SemaphoreType.DMA((NBUF, 2)),
        ],
        out_shape=jax.ShapeDtypeStruct((B, S, D), jnp.float32),
    )(hidden_states, ln1_w.reshape(1, D), wq, wk, wv,
      q_norm_w.reshape(1, DH), k_norm_w.reshape(1, DH), wo,
      ln2_w.reshape(1, D), router_w, w_gate_up, w_down)
    return out


# speedup vs baseline: 1.1983x; 1.1864x over previous
import math

import jax
import jax.numpy as jnp
from jax.experimental import pallas as pl
from jax.experimental.pallas import tpu as pltpu

D = 1024
H = 16
HK = 4
DH = 64
E = 16
F = 768
B = 32
S = 4
T = B * S
EPS = 1e-06
THETA = 1000000.0
TILE = 16
TP = 256
NBUF = 4


def _rms(x, w, eps=EPS):
    var = jnp.mean(x * x, axis=-1, keepdims=True)
    return x * jax.lax.rsqrt(var + eps) * w


def _fused_body(h_ref, ln1_ref, wq_ref, wk_ref, wv_ref, qn_ref, kn_ref,
                wo_ref, ln2_ref, rw_ref, wgu_hbm, wd_hbm, out_ref,
                x2_scr, p_scr, off_scr, cnt_scr, y_scr,
                gu_buf, d_buf, gu_sem, d_sem):
    i = pl.program_id(0)

    def _gu_copy(idx, s, half):
        rows = pl.ds(half * (D // 2), D // 2)
        return pltpu.make_async_copy(wgu_hbm.at[idx, rows, :],
                                     gu_buf.at[s, rows, :],
                                     gu_sem.at[s, half])

    def _d_copy(idx, s, half):
        rows = pl.ds(half * (F // 2), F // 2)
        return pltpu.make_async_copy(wd_hbm.at[idx, rows, :],
                                     d_buf.at[s, rows, :],
                                     d_sem.at[s, half])

    def start_copy(idx, s):
        for half in range(2):
            _gu_copy(idx, s, half).start()
            _d_copy(idx, s, half).start()

    @pl.when(i == 0)
    def _attn_router():
        for b in range(NBUF):
            start_copy(b, b)
        y_scr[...] = jnp.zeros_like(y_scr)

        h = h_ref[...].reshape(T, D)
        x = _rms(h, ln1_ref[...])
        q2 = jnp.dot(x, wq_ref[...], preferred_element_type=jnp.float32)
        k2 = jnp.dot(x, wk_ref[...], preferred_element_type=jnp.float32)
        v2 = jnp.dot(x, wv_ref[...], preferred_element_type=jnp.float32)
        rowpos = jax.lax.broadcasted_iota(jnp.int32, (T, DH), 0) % S
        colf = jax.lax.broadcasted_iota(jnp.int32, (T, DH), 1) % (DH // 2)
        inv = jnp.exp(colf.astype(jnp.float32) *
                      jnp.float32(-2.0 * math.log(THETA) / DH))
        ang = rowpos.astype(jnp.float32) * inv
        cos = jnp.cos(ang)
        sin = jnp.sin(ang)
        qn = qn_ref[...]
        kn = kn_ref[...]

        def rope(z):
            zr = jnp.concatenate([-z[:, DH // 2:], z[:, :DH // 2]], axis=1)
            return z * cos + zr * sin

        ks = []
        vs = []
        for j in range(HK):
            kj = rope(_rms(k2[:, j * DH:(j + 1) * DH], kn))
            ks.append(kj)
            vs.append(v2[:, j * DH:(j + 1) * DH])

        ti = jax.lax.broadcasted_iota(jnp.int32, (T, T), 0)
        tj = jax.lax.broadcasted_iota(jnp.int32, (T, T), 1)
        mask = (ti // S == tj // S) & (tj % S <= ti % S)
        neg = jnp.float32(-1e30)
        scale = jnp.float32(1.0 / math.sqrt(DH))

        outs = []
        for hh in range(H):
            j = hh // (H // HK)
            qh = rope(_rms(q2[:, hh * DH:(hh + 1) * DH], qn))
            sc = jax.lax.dot_general(qh, ks[j], (((1,), (1,)), ((), ())),
                                     preferred_element_type=jnp.float32)
            sc = jnp.where(mask, sc * scale, neg)
            ex = jnp.exp(sc - jnp.max(sc, axis=1, keepdims=True))
            at = ex / jnp.sum(ex, axis=1, keepdims=True)
            outs.append(jnp.dot(at, vs[j], preferred_element_type=jnp.float32))
        o2 = jnp.concatenate(outs, axis=1)
        hidden = h + jnp.dot(o2, wo_ref[...],
                             preferred_element_type=jnp.float32)
        out_ref[...] = hidden.reshape(B, S, D)

        x2 = _rms(hidden, ln2_ref[...])
        logits = jnp.dot(x2, rw_ref[...], preferred_element_type=jnp.float32)

        eidx = jax.lax.broadcasted_iota(jnp.int32, (T, E), 1)
        rowmax = jnp.max(logits, axis=1, keepdims=True)
        assign = jnp.min(jnp.where(logits == rowmax, eidx, E), axis=1,
                         keepdims=True)
        onehot = (eidx == assign).astype(jnp.float32)
        counts = jnp.sum(onehot, axis=0, keepdims=True)
        padded = jnp.float32(8.0) * jnp.ceil(counts * jnp.float32(1.0 / 8.0))
        er = jax.lax.broadcasted_iota(jnp.int32, (E, E), 0)
        ec = jax.lax.broadcasted_iota(jnp.int32, (E, E), 1)
        lt_e = (er < ec).astype(jnp.float32)
        offsets = jnp.dot(padded, lt_e,
                          preferred_element_type=jnp.float32)
        lt_t = (tj < ti).astype(jnp.float32)
        intra = jnp.dot(lt_t, onehot,
                        preferred_element_type=jnp.float32)
        rank = jnp.sum(onehot * (offsets + intra), axis=1,
                       keepdims=True)
        rj = jax.lax.broadcasted_iota(jnp.int32, (T, TP), 1)
        p_mat = (rank.astype(jnp.int32) == rj).astype(jnp.float32)
        p_scr[...] = p_mat
        x2_scr[...] = jnp.dot(p_mat.T, x2, preferred_element_type=jnp.float32)
        se = jax.lax.broadcasted_iota(jnp.int32, (E, E * 128), 0)
        sl = jax.lax.broadcasted_iota(jnp.int32, (E, E * 128), 1)
        spread = (sl // 128 == se).astype(jnp.float32)
        off_scr[...] = jnp.dot(offsets * jnp.float32(1.0 / 8.0), spread,
                               preferred_element_type=jnp.float32
                               ).astype(jnp.int32)
        cnt_scr[...] = jnp.dot(counts, spread,
                               preferred_element_type=jnp.float32
                               ).astype(jnp.int32)

    @pl.when(i > 0)
    def _expert():
        e = i - 1
        slot = jax.lax.rem(e, NBUF)
        off8 = off_scr[0, e * 128]
        cnt = cnt_scr[0, e * 128]
        for half in range(2):
            _gu_copy(e, slot, half).wait()
            _d_copy(e, slot, half).wait()

        ntiles = (cnt + TILE - 1) // TILE
        wgu = gu_buf[slot]
        wd = d_buf[slot]
        rid = jax.lax.broadcasted_iota(jnp.int32, (TILE, 1), 0)

        def tile_step(t, _):
            base = off8 * 8 + t * TILE
            xb = x2_scr[pl.ds(base, TILE), :]
            gu = jnp.dot(xb, wgu, preferred_element_type=jnp.float32)
            gate = gu[:, :F]
            up = gu[:, F:]
            act = gate * jax.lax.logistic(gate) * up
            y = jnp.dot(act, wd, preferred_element_type=jnp.float32)
            valid = t * TILE + rid < cnt
            cur = y_scr[pl.ds(base, TILE), :]
            y_scr[pl.ds(base, TILE), :] = jnp.where(valid, y, cur)
            return 0

        jax.lax.fori_loop(0, ntiles, tile_step, 0)

        @pl.when(e + NBUF < E)
        def _next():
            start_copy(e + NBUF, slot)

    @pl.when(i == E)
    def _finish():
        moe = jnp.dot(p_scr[...], y_scr[...],
                      preferred_element_type=jnp.float32)
        out_ref[...] = out_ref[...] + moe.reshape(B, S, D)


@jax.jit
def kernel(hidden_states, ln1_w, wq, wk, wv, q_norm_w, k_norm_w, wo, ln2_w,
           router_w, w_gate_up, w_down):
    whole = lambda e: (0,) * 2
    out = pl.pallas_call(
        _fused_body,
        grid=(E + 1,),
        in_specs=[
            pl.BlockSpec((B, S, D), lambda e: (0, 0, 0)),
            pl.BlockSpec((1, D), whole),
            pl.BlockSpec((D, H * DH), whole),
            pl.BlockSpec((D, HK * DH), whole),
            pl.BlockSpec((D, HK * DH), whole),
            pl.BlockSpec((1, DH), whole),
            pl.BlockSpec((1, DH), whole),
            pl.BlockSpec((H * DH, D), whole),
            pl.BlockSpec((1, D), whole),
            pl.BlockSpec((D, E), whole),
            pl.BlockSpec(memory_space=pltpu.MemorySpace.HBM),
            pl.BlockSpec(memory_space=pltpu.MemorySpace.HBM),
        ],
        out_specs=pl.BlockSpec((B, S, D), lambda e: (0, 0, 0)),
        scratch_shapes=[
            pltpu.VMEM((TP, D), jnp.float32),
            pltpu.VMEM((T, TP), jnp.float32),
            pltpu.VMEM((1, E * 128), jnp.int32),
            pltpu.VMEM((1, E * 128), jnp.int32),
            pltpu.VMEM((TP, D), jnp.float32),
            pltpu.VMEM((NBUF, D, 2 * F), jnp.float32),
            pltpu.VMEM((NBUF, F, D), jnp.float32),
            pltpu.SemaphoreType.DMA((NBUF, 2)),
            pltpu.SemaphoreType.DMA((NBUF, 2)),
        ],
        out_shape=jax.ShapeDtypeStruct((B, S, D), jnp.float32),
    )(hidden_states, ln1_w.reshape(1, D), wq, wk, wv,
      q_norm_w.reshape(1, DH), k_norm_w.reshape(1, DH), wo,
      ln2_w.reshape(1, D), router_w, w_gate_up, w_down)
    return out
